# Initial kernel scaffold; baseline (speedup 1.0000x reference)
import jax
import jax.numpy as jnp
from jax.experimental import pallas as pl


def kernel(states, speaker_ids, emotion_ids, embed, W_ih, W_hh, b_ih, b_hh):
    raise NotImplementedError("write your pallas kernel here")


# speedup vs baseline: 7.4528x; 7.4528x over previous
import jax
import jax.numpy as jnp
from jax.experimental import pallas as pl
from jax.experimental.pallas import tpu as pltpu

_S = 16
_H = 128
_E = 64
_NEMO = 32
_BR = 256


def _fused_body(idx_ref, emo_ref, states_ref, embed_ref, wih_ref, whh_ref,
                bih_ref, bhh_ref, out_ref):
    idx = idx_ref[...]
    emo = emo_ref[...]

    safe = jnp.where(emo >= 0, emo, _NEMO)
    cols = jax.lax.broadcasted_iota(jnp.int32, (1, _NEMO + 1), 1)
    onehot = (safe == cols).astype(jnp.float32)
    emb = jnp.dot(onehot, embed_ref[...],
                  preferred_element_type=jnp.float32)

    h_old = jnp.zeros((_BR, _H), jnp.float32)
    for s in range(_S):
        h_old = h_old + jnp.where(idx == s, states_ref[:, s, :], 0.0)

    gi = jnp.dot(emb, wih_ref[...],
                 preferred_element_type=jnp.float32) + bih_ref[...]
    gh = jnp.dot(h_old, whh_ref[...],
                 preferred_element_type=jnp.float32) + bhh_ref[...]
    r = jax.nn.sigmoid(gi[:, :_H] + gh[:, :_H])
    z = jax.nn.sigmoid(gi[:, _H:2 * _H] + gh[:, _H:2 * _H])
    n = jnp.tanh(gi[:, 2 * _H:] + r * gh[:, 2 * _H:])
    h_new = (1.0 - z) * n + z * h_old

    for s in range(_S):
        out_ref[:, s, :] = jnp.where(idx == s, h_new, states_ref[:, s, :])


def kernel(states, speaker_ids, emotion_ids, embed, W_ih, W_hh, b_ih, b_hh):
    B, S, H = states.shape
    nb = B // _BR
    idx = jnp.minimum(speaker_ids.astype(jnp.int32), S - 1).reshape(B, 1)
    emo = emotion_ids.astype(jnp.int32).reshape(B, 1)

    grid_spec = pl.GridSpec(
        grid=(nb,),
        in_specs=[
            pl.BlockSpec((_BR, 1), lambda i: (i, 0)),
            pl.BlockSpec((_BR, 1), lambda i: (i, 0)),
            pl.BlockSpec((_BR, S, H), lambda i: (i, 0, 0)),
            pl.BlockSpec((_NEMO + 1, _E), lambda i: (0, 0)),
            pl.BlockSpec((_E, 3 * _H), lambda i: (0, 0)),
            pl.BlockSpec((_H, 3 * _H), lambda i: (0, 0)),
            pl.BlockSpec((1, 3 * _H), lambda i: (0, 0)),
            pl.BlockSpec((1, 3 * _H), lambda i: (0, 0)),
        ],
        out_specs=pl.BlockSpec((_BR, S, H), lambda i: (i, 0, 0)),
    )
    return pl.pallas_call(
        _fused_body,
        grid_spec=grid_spec,
        out_shape=jax.ShapeDtypeStruct((B, S, H), states.dtype),
        compiler_params=pltpu.CompilerParams(
            dimension_semantics=("arbitrary",),
        ),
    )(idx, emo, states, embed, W_ih.T, W_hh.T,
      b_ih.reshape(1, -1), b_hh.reshape(1, -1))
